# baseline (speedup 1.0000x reference)
import functools

import jax
import jax.numpy as jnp
from jax import lax
from jax.experimental import pallas as pl
from jax.experimental.pallas import tpu as pltpu
from jax.experimental.pallas import tpu_sc as plsc

B, N, D = 64, 576, 768
HID, CD, CS = 512, 256, 512
M = B * N
TILE = 2048
GRID = M // TILE

NC, NS = 2, 16
NW = NC * NS
BPW = M // NW
CH = 64
NCHUNK = BPW // CH


def _precompute_kernel(cb_ref, wd1_ref, bd1_ref, wd2_ref, bd2_ref,
                       dec_cb_ref, c2_ref):
    hd = jnp.maximum(
        jnp.dot(cb_ref[...], wd1_ref[...],
                preferred_element_type=jnp.float32) + bd1_ref[...], 0.0)
    dec_cb_ref[...] = jnp.dot(hd, wd2_ref[...],
                              preferred_element_type=jnp.float32) + bd2_ref[...]
    c2_ref[...] = jnp.sum(cb_ref[...] * cb_ref[...], axis=-1)[None, :]


def _encode_kernel(x_ref, w1_ref, b1_ref, w2_ref, b2_ref, cb_ref, c2_ref,
                   idx_ref, loss_ref):
    step = pl.program_id(0)

    @pl.when(step == 0)
    def _():
        loss_ref[...] = jnp.zeros((1, 1), jnp.float32)

    h = jnp.maximum(
        jnp.dot(x_ref[...], w1_ref[...],
                preferred_element_type=jnp.float32) + b1_ref[...], 0.0)
    z = jnp.dot(h, w2_ref[...],
                preferred_element_type=jnp.float32) + b2_ref[...]

    z2 = jnp.sum(z * z, axis=-1, keepdims=True)
    zc = lax.dot_general(z, cb_ref[...],
                         (((1,), (1,)), ((), ())),
                         preferred_element_type=jnp.float32)
    dist = z2 - 2.0 * zc + c2_ref[...]

    idx_ref[...] = jnp.argmin(dist, axis=-1).astype(jnp.int32)[:, None]
    loss_ref[...] += jnp.sum(jnp.min(dist, axis=-1)).reshape(1, 1)


def _sc_gather_kernel(idx_hbm, table_hbm, out_hbm, idx_v, bufs, sem0, sem1):
    sems = (sem0, sem1)
    wid = lax.axis_index("s") * NC + lax.axis_index("c")
    base = wid * BPW
    pltpu.sync_copy(idx_hbm.at[pl.ds(base, BPW)], idx_v)

    def start(c):
        return pltpu.async_copy(
            table_hbm.at[idx_v.at[pl.ds(c * CH, CH)]],
            bufs.at[c % 2], sems[c % 2])

    copies = [start(0), None]
    for c in range(NCHUNK):
        if c + 1 < NCHUNK:
            copies[(c + 1) % 2] = start(c + 1)
        copies[c % 2].wait()
        pltpu.sync_copy(bufs.at[c % 2], out_hbm.at[pl.ds(base + c * CH, CH)])


_sc_gather = pl.kernel(
    _sc_gather_kernel,
    mesh=plsc.VectorSubcoreMesh(core_axis_name="c", subcore_axis_name="s"),
    out_type=jax.ShapeDtypeStruct((M, D), jnp.float32),
    scratch_types=[
        pltpu.VMEM((BPW,), jnp.int32),
        pltpu.VMEM((2, CH, D), jnp.float32),
        pltpu.SemaphoreType.DMA,
        pltpu.SemaphoreType.DMA,
    ],
)


@jax.jit
def kernel(x, W1, b1, W2, b2, codebook, Wd1, bd1, Wd2, bd2):
    x2 = x.reshape(M, D)
    full = lambda shape: pl.BlockSpec(shape, lambda *_: (0,) * len(shape))

    dec_cb, c2 = pl.pallas_call(
        _precompute_kernel,
        out_shape=[
            jax.ShapeDtypeStruct((CS, D), jnp.float32),
            jax.ShapeDtypeStruct((1, CS), jnp.float32),
        ],
    )(codebook, Wd1, bd1.reshape(1, HID), Wd2, bd2.reshape(1, D))

    idx, loss = pl.pallas_call(
        _encode_kernel,
        grid=(GRID,),
        in_specs=[
            pl.BlockSpec((TILE, D), lambda i: (i, 0)),
            full((D, HID)), full((1, HID)),
            full((HID, CD)), full((1, CD)),
            full((CS, CD)), full((1, CS)),
        ],
        out_specs=[
            pl.BlockSpec((TILE, 1), lambda i: (i, 0)),
            pl.BlockSpec((1, 1), lambda i: (0, 0)),
        ],
        out_shape=[
            jax.ShapeDtypeStruct((M, 1), jnp.int32),
            jax.ShapeDtypeStruct((1, 1), jnp.float32),
        ],
        compiler_params=pltpu.CompilerParams(
            dimension_semantics=("arbitrary",)),
    )(x2, W1, b1.reshape(1, HID), W2, b2.reshape(1, CD), codebook, c2)

    recon = _sc_gather(idx.reshape(M), dec_cb).reshape(B, N, D)
    indices = idx.reshape(B, N)
    commit_loss = loss[0, 0] / (M * CD)
    return recon, indices, commit_loss


# speedup vs baseline: 1.0072x; 1.0072x over previous
import functools

import jax
import jax.numpy as jnp
from jax import lax
from jax.experimental import pallas as pl
from jax.experimental.pallas import tpu as pltpu
from jax.experimental.pallas import tpu_sc as plsc

B, N, D = 64, 576, 768
HID, CD, CS = 512, 256, 512
M = B * N
TILE = 2048
GRID = M // TILE

NC, NS = 2, 16
NW = NC * NS
BPW = M // NW
CH = 128
NCHUNK = BPW // CH


def _precompute_kernel(cb_ref, wd1_ref, bd1_ref, wd2_ref, bd2_ref,
                       dec_cb_ref, c2_ref):
    hd = jnp.maximum(
        jnp.dot(cb_ref[...], wd1_ref[...],
                preferred_element_type=jnp.float32) + bd1_ref[...], 0.0)
    dec_cb_ref[...] = jnp.dot(hd, wd2_ref[...],
                              preferred_element_type=jnp.float32) + bd2_ref[...]
    c2_ref[...] = jnp.sum(cb_ref[...] * cb_ref[...], axis=-1)[None, :]


def _encode_kernel(x_ref, w1_ref, b1_ref, w2_ref, b2_ref, cb_ref, c2_ref,
                   idx_ref, loss_ref):
    step = pl.program_id(0)

    @pl.when(step == 0)
    def _():
        loss_ref[...] = jnp.zeros((1, 1), jnp.float32)

    h = jnp.maximum(
        jnp.dot(x_ref[...], w1_ref[...],
                preferred_element_type=jnp.float32) + b1_ref[...], 0.0)
    z = jnp.dot(h, w2_ref[...],
                preferred_element_type=jnp.float32) + b2_ref[...]

    z2 = jnp.sum(z * z, axis=-1, keepdims=True)
    zc = lax.dot_general(z, cb_ref[...],
                         (((1,), (1,)), ((), ())),
                         preferred_element_type=jnp.float32)
    dist = z2 - 2.0 * zc + c2_ref[...]

    idx_ref[...] = jnp.argmin(dist, axis=-1).astype(jnp.int32)[:, None]
    loss_ref[...] += jnp.sum(jnp.min(dist, axis=-1)).reshape(1, 1)


def _sc_gather_kernel(idx_hbm, table_hbm, out_hbm, idx_v, bufs, sem0):
    wid = lax.axis_index("s") * NC + lax.axis_index("c")
    base = wid * BPW
    pltpu.sync_copy(idx_hbm.at[pl.ds(base, BPW)], idx_v)

    for c in range(NCHUNK):
        pltpu.async_copy(
            table_hbm.at[idx_v.at[pl.ds(c * CH, CH)]], bufs, sem0).wait()
        pltpu.sync_copy(bufs, out_hbm.at[pl.ds(base + c * CH, CH)])


_sc_gather = pl.kernel(
    _sc_gather_kernel,
    mesh=plsc.VectorSubcoreMesh(core_axis_name="c", subcore_axis_name="s"),
    out_type=jax.ShapeDtypeStruct((M, D), jnp.float32),
    scratch_types=[
        pltpu.VMEM((BPW,), jnp.int32),
        pltpu.VMEM((CH, D), jnp.float32),
        pltpu.SemaphoreType.DMA,
    ],
)


@jax.jit
def kernel(x, W1, b1, W2, b2, codebook, Wd1, bd1, Wd2, bd2):
    x2 = x.reshape(M, D)
    full = lambda shape: pl.BlockSpec(shape, lambda *_: (0,) * len(shape))

    dec_cb, c2 = pl.pallas_call(
        _precompute_kernel,
        out_shape=[
            jax.ShapeDtypeStruct((CS, D), jnp.float32),
            jax.ShapeDtypeStruct((1, CS), jnp.float32),
        ],
    )(codebook, Wd1, bd1.reshape(1, HID), Wd2, bd2.reshape(1, D))

    idx, loss = pl.pallas_call(
        _encode_kernel,
        grid=(GRID,),
        in_specs=[
            pl.BlockSpec((TILE, D), lambda i: (i, 0)),
            full((D, HID)), full((1, HID)),
            full((HID, CD)), full((1, CD)),
            full((CS, CD)), full((1, CS)),
        ],
        out_specs=[
            pl.BlockSpec((TILE, 1), lambda i: (i, 0)),
            pl.BlockSpec((1, 1), lambda i: (0, 0)),
        ],
        out_shape=[
            jax.ShapeDtypeStruct((M, 1), jnp.int32),
            jax.ShapeDtypeStruct((1, 1), jnp.float32),
        ],
        compiler_params=pltpu.CompilerParams(
            dimension_semantics=("arbitrary",)),
    )(x2, W1, b1.reshape(1, HID), W2, b2.reshape(1, CD), codebook, c2)

    recon = _sc_gather(idx.reshape(M), dec_cb).reshape(B, N, D)
    indices = idx.reshape(B, N)
    commit_loss = loss[0, 0] / (M * CD)
    return recon, indices, commit_loss


# speedup vs baseline: 6.0892x; 6.0459x over previous
import jax
import jax.numpy as jnp
from jax import lax
from jax.experimental import pallas as pl
from jax.experimental.pallas import tpu as pltpu

B, N, D = 64, 576, 768
HID, CD, CS = 512, 256, 512
M = B * N
TILE = 2048
GRID = M // TILE


def _fused_kernel(x_ref, w1_ref, b1_ref, w2_ref, b2_ref, cb_ref,
                  wd1_ref, bd1_ref, wd2_ref, bd2_ref,
                  recon_ref, idx_ref, loss_ref,
                  dec_cb_ref, c2_ref):
    step = pl.program_id(0)

    @pl.when(step == 0)
    def _():
        hd = jnp.maximum(
            jnp.dot(cb_ref[...], wd1_ref[...],
                    preferred_element_type=jnp.float32) + bd1_ref[...], 0.0)
        dec = jnp.dot(hd, wd2_ref[...],
                      preferred_element_type=jnp.float32) + bd2_ref[...]
        dec_cb_ref[...] = dec.astype(jnp.bfloat16)
        c2_ref[...] = jnp.sum(cb_ref[...] * cb_ref[...], axis=-1)[None, :]
        loss_ref[...] = jnp.zeros((1, 1), jnp.float32)

    h = jnp.maximum(
        jnp.dot(x_ref[...], w1_ref[...],
                preferred_element_type=jnp.float32) + b1_ref[...], 0.0)
    z = jnp.dot(h, w2_ref[...],
                preferred_element_type=jnp.float32) + b2_ref[...]

    z2 = jnp.sum(z * z, axis=-1, keepdims=True)
    zc = lax.dot_general(z, cb_ref[...],
                         (((1,), (1,)), ((), ())),
                         preferred_element_type=jnp.float32)
    dist = z2 - 2.0 * zc + c2_ref[...]

    idx = jnp.argmin(dist, axis=-1).astype(jnp.int32)
    idx_ref[...] = idx[:, None]
    loss_ref[...] += jnp.sum(jnp.min(dist, axis=-1)).reshape(1, 1)

    onehot = (idx[:, None] == lax.broadcasted_iota(jnp.int32, (1, CS), 1)
              ).astype(jnp.bfloat16)
    recon_ref[...] = jnp.dot(onehot, dec_cb_ref[...],
                             preferred_element_type=jnp.float32)


@jax.jit
def kernel(x, W1, b1, W2, b2, codebook, Wd1, bd1, Wd2, bd2):
    x2 = x.reshape(M, D)
    full = lambda shape: pl.BlockSpec(shape, lambda *_: (0,) * len(shape))

    recon, idx, loss = pl.pallas_call(
        _fused_kernel,
        grid=(GRID,),
        in_specs=[
            pl.BlockSpec((TILE, D), lambda i: (i, 0)),
            full((D, HID)), full((1, HID)),
            full((HID, CD)), full((1, CD)),
            full((CS, CD)),
            full((CD, HID)), full((1, HID)),
            full((HID, D)), full((1, D)),
        ],
        out_specs=[
            pl.BlockSpec((TILE, D), lambda i: (i, 0)),
            pl.BlockSpec((TILE, 1), lambda i: (i, 0)),
            pl.BlockSpec((1, 1), lambda i: (0, 0)),
        ],
        out_shape=[
            jax.ShapeDtypeStruct((M, D), jnp.float32),
            jax.ShapeDtypeStruct((M, 1), jnp.int32),
            jax.ShapeDtypeStruct((1, 1), jnp.float32),
        ],
        scratch_shapes=[
            pltpu.VMEM((CS, D), jnp.bfloat16),
            pltpu.VMEM((1, CS), jnp.float32),
        ],
        compiler_params=pltpu.CompilerParams(
            dimension_semantics=("arbitrary",)),
    )(x2, W1, b1.reshape(1, HID), W2, b2.reshape(1, CD), codebook,
      Wd1, bd1.reshape(1, HID), Wd2, bd2.reshape(1, D))

    recon = recon.reshape(B, N, D)
    indices = idx.reshape(B, N)
    commit_loss = loss[0, 0] / (M * CD)
    return recon, indices, commit_loss
